# baseline (speedup 1.0000x reference)
import jax
import jax.numpy as jnp
from jax import lax
from jax.experimental import pallas as pl
from jax.experimental.pallas import tpu as pltpu
from jax.experimental.pallas import tpu_sc as plsc

NUM_PATCHES = 1024
D = 768
B = 64
NUM_WORKERS = 32
F_PER_W = D // NUM_WORKERS
LANES = 16
VECS_PER_ROW = D // LANES
IDX_PER_W = B * F_PER_W


NBUF = 2


def _emb_body(x_hbm, tok_hbm, pos_hbm, out_hbm,
              pos_v, idx_all, tok_sp, rows0, rows1,
              g0, g1, s0, s1):
    c = lax.axis_index("c")
    s = lax.axis_index("s")
    wid = s * 2 + c
    f0 = wid * F_PER_W

    rows = (rows0, rows1)
    gsem = (g0, g1)
    ssem = (s0, s1)

    rows_per_tile = NUM_PATCHES // 16
    pltpu.sync_copy(tok_hbm.at[pl.ds(s * rows_per_tile, rows_per_tile)],
                    tok_sp.at[pl.ds(s * rows_per_tile, rows_per_tile)])

    pltpu.sync_copy(pos_hbm.at[pl.ds(f0, F_PER_W)], pos_v)
    pltpu.sync_copy(x_hbm.at[pl.ds(wid * IDX_PER_W, IDX_PER_W)], idx_all)

    plsc.subcore_barrier()

    def out_slice(bb):
        return out_hbm.at[pl.ds(bb * D + f0, F_PER_W)]

    def launch_gather(bb, k):
        base = bb * F_PER_W
        v0 = idx_all[pl.ds(base, LANES)]
        v1 = idx_all[pl.ds(base + 8, LANES)]
        for r in range(F_PER_W):
            xv = v0[r] if r < LANES else v1[r - 8]
            pltpu.async_copy(tok_sp.at[pl.ds(xv, 1)],
                             rows[k].at[pl.ds(r, 1)], gsem[k])

    def wait_gather(k):
        pltpu.make_async_copy(tok_sp.at[pl.ds(0, F_PER_W)], rows[k],
                              gsem[k]).wait()

    for k in range(NBUF - 1):
        launch_gather(k, k)

    def step(i, k):
        bb = NBUF * i + k
        cur = rows[k]
        prv = rows[(k + NBUF - 1) % NBUF]

        @pl.when(bb >= 1)
        def _():
            pltpu.make_async_copy(
                prv, out_slice(bb - 1), ssem[(k + NBUF - 1) % NBUF]).wait()

        @pl.when(bb + NBUF - 1 < B)
        def _():
            launch_gather(bb + NBUF - 1, (k + NBUF - 1) % NBUF)

        wait_gather(k)

        @pl.loop(0, F_PER_W)
        def _(r):
            for j in range(VECS_PER_ROW):
                sl = pl.ds(j * LANES, LANES)
                plsc.addupdate(cur.at[r, sl], pos_v[r, sl])

        pltpu.async_copy(cur, out_slice(bb), ssem[k])

    def body(i, carry):
        for k in range(NBUF):
            step(i, k)
        return carry

    lax.fori_loop(0, B // NBUF, body, 0)
    pltpu.make_async_copy(rows[(B - 1) % NBUF], out_slice(B - 1),
                          ssem[(B - 1) % NBUF]).wait()


@jax.jit
def kernel(x, token_table, pos_table):
    xp = x.reshape(B, NUM_WORKERS, F_PER_W).transpose(1, 0, 2).reshape(-1)
    mesh = plsc.VectorSubcoreMesh(core_axis_name="c", subcore_axis_name="s")
    out = pl.kernel(
        _emb_body,
        out_type=jax.ShapeDtypeStruct((B * D, D), jnp.float32),
        mesh=mesh,
        scratch_types=[
            pltpu.VMEM((F_PER_W, D), jnp.float32),
            pltpu.VMEM((IDX_PER_W,), jnp.int32),
            pltpu.VMEM_SHARED((NUM_PATCHES, D), jnp.float32),
            pltpu.VMEM((F_PER_W, D), jnp.float32),
            pltpu.VMEM((F_PER_W, D), jnp.float32),
            pltpu.SemaphoreType.DMA,
            pltpu.SemaphoreType.DMA,
            pltpu.SemaphoreType.DMA,
            pltpu.SemaphoreType.DMA,
        ],
    )(xp, token_table, pos_table)
    return out.reshape(B, D, D)


# speedup vs baseline: 1.0030x; 1.0030x over previous
import jax
import jax.numpy as jnp
from jax import lax
from jax.experimental import pallas as pl
from jax.experimental.pallas import tpu as pltpu
from jax.experimental.pallas import tpu_sc as plsc

NUM_PATCHES = 1024
D = 768
B = 64
NUM_WORKERS = 32
F_PER_W = D // NUM_WORKERS
LANES = 16
VECS_PER_ROW = D // LANES
IDX_PER_W = B * F_PER_W


NBUF = 2
H = 8


def _emb_body(x_hbm, tok_hbm, pos_hbm, out_hbm,
              pos_v, idx_all, tok_sp, rows0, rows1,
              g0, g1, s0, s1):
    c = lax.axis_index("c")
    s = lax.axis_index("s")
    wid = s * 2 + c
    f0 = wid * F_PER_W

    rows = (rows0, rows1)
    gsem = (g0, g1)
    ssem = (s0, s1)

    rows_per_tile = NUM_PATCHES // 16
    pltpu.sync_copy(tok_hbm.at[pl.ds(s * rows_per_tile, rows_per_tile)],
                    tok_sp.at[pl.ds(s * rows_per_tile, rows_per_tile)])

    pltpu.sync_copy(pos_hbm.at[pl.ds(f0, F_PER_W)], pos_v)
    pltpu.sync_copy(x_hbm.at[pl.ds(wid * IDX_PER_W, IDX_PER_W)], idx_all)

    plsc.subcore_barrier()

    def out_slice(bb):
        return out_hbm.at[pl.ds(bb * D + f0, F_PER_W)]

    def launch_gather(bb, k):
        base = bb * F_PER_W
        pltpu.async_copy(tok_hbm.at[idx_all.at[pl.ds(base, H)]],
                         rows[k].at[pl.ds(0, H)], gsem[k])
        v0 = idx_all[pl.ds(base, LANES)]
        v1 = idx_all[pl.ds(base + 8, LANES)]
        for r in range(H, F_PER_W):
            xv = v0[r] if r < LANES else v1[r - 8]
            pltpu.async_copy(tok_sp.at[pl.ds(xv, 1)],
                             rows[k].at[pl.ds(r, 1)], gsem[k])

    def wait_gather(k):
        pltpu.make_async_copy(tok_sp.at[pl.ds(0, F_PER_W)], rows[k],
                              gsem[k]).wait()

    for k in range(NBUF - 1):
        launch_gather(k, k)

    def step(i, k):
        bb = NBUF * i + k
        cur = rows[k]
        prv = rows[(k + NBUF - 1) % NBUF]

        @pl.when(bb >= 1)
        def _():
            pltpu.make_async_copy(
                prv, out_slice(bb - 1), ssem[(k + NBUF - 1) % NBUF]).wait()

        @pl.when(bb + NBUF - 1 < B)
        def _():
            launch_gather(bb + NBUF - 1, (k + NBUF - 1) % NBUF)

        wait_gather(k)

        @pl.loop(0, F_PER_W)
        def _(r):
            for j in range(VECS_PER_ROW):
                sl = pl.ds(j * LANES, LANES)
                plsc.addupdate(cur.at[r, sl], pos_v[r, sl])

        pltpu.async_copy(cur, out_slice(bb), ssem[k])

    def body(i, carry):
        for k in range(NBUF):
            step(i, k)
        return carry

    lax.fori_loop(0, B // NBUF, body, 0)
    pltpu.make_async_copy(rows[(B - 1) % NBUF], out_slice(B - 1),
                          ssem[(B - 1) % NBUF]).wait()


@jax.jit
def kernel(x, token_table, pos_table):
    xp = x.reshape(B, NUM_WORKERS, F_PER_W).transpose(1, 0, 2).reshape(-1)
    mesh = plsc.VectorSubcoreMesh(core_axis_name="c", subcore_axis_name="s")
    out = pl.kernel(
        _emb_body,
        out_type=jax.ShapeDtypeStruct((B * D, D), jnp.float32),
        mesh=mesh,
        scratch_types=[
            pltpu.VMEM((F_PER_W, D), jnp.float32),
            pltpu.VMEM((IDX_PER_W,), jnp.int32),
            pltpu.VMEM_SHARED((NUM_PATCHES, D), jnp.float32),
            pltpu.VMEM((F_PER_W, D), jnp.float32),
            pltpu.VMEM((F_PER_W, D), jnp.float32),
            pltpu.SemaphoreType.DMA,
            pltpu.SemaphoreType.DMA,
            pltpu.SemaphoreType.DMA,
            pltpu.SemaphoreType.DMA,
        ],
    )(xp, token_table, pos_table)
    return out.reshape(B, D, D)


# speedup vs baseline: 1.0709x; 1.0677x over previous
import jax
import jax.numpy as jnp
from jax import lax
from jax.experimental import pallas as pl
from jax.experimental.pallas import tpu as pltpu
from jax.experimental.pallas import tpu_sc as plsc

NUM_PATCHES = 1024
D = 768
B = 64
NUM_WORKERS = 32
F_PER_W = D // NUM_WORKERS
LANES = 16
VECS_PER_ROW = D // LANES
IDX_PER_W = B * F_PER_W


NBUF = 2
H = 8


def _emb_body(x_hbm, tok_hbm, pos_hbm, out_hbm,
              pos_v, idx_all, tok_sp, rows0, rows1,
              g0, g1, h0, h1, s0, s1):
    c = lax.axis_index("c")
    s = lax.axis_index("s")
    wid = s * 2 + c
    f0 = wid * F_PER_W

    rows = (rows0, rows1)
    gsem = (g0, g1)
    hsem = (h0, h1)
    ssem = (s0, s1)

    rows_per_tile = NUM_PATCHES // 16
    pltpu.sync_copy(tok_hbm.at[pl.ds(s * rows_per_tile, rows_per_tile)],
                    tok_sp.at[pl.ds(s * rows_per_tile, rows_per_tile)])

    pltpu.sync_copy(pos_hbm.at[pl.ds(f0, F_PER_W)], pos_v)
    pltpu.sync_copy(x_hbm.at[pl.ds(wid * IDX_PER_W, IDX_PER_W)], idx_all)

    plsc.subcore_barrier()

    def out_slice(bb):
        return out_hbm.at[pl.ds(bb * D + f0, F_PER_W)]

    def launch_gather(bb, k):
        base = bb * F_PER_W
        pltpu.async_copy(tok_hbm.at[idx_all.at[pl.ds(base, H)]],
                         rows[k].at[pl.ds(0, H)], hsem[k])
        v0 = idx_all[pl.ds(base, LANES)]
        v1 = idx_all[pl.ds(base + 8, LANES)]
        for r in range(H, F_PER_W):
            xv = v0[r] if r < LANES else v1[r - 8]
            pltpu.async_copy(tok_sp.at[pl.ds(xv, 1)],
                             rows[k].at[pl.ds(r, 1)], gsem[k])

    def wait_gather(k):
        pltpu.make_async_copy(tok_hbm.at[idx_all.at[pl.ds(0, H)]],
                              rows[k].at[pl.ds(0, H)], hsem[k]).wait()
        pltpu.make_async_copy(tok_sp.at[pl.ds(0, F_PER_W - H)],
                              rows[k].at[pl.ds(H, F_PER_W - H)],
                              gsem[k]).wait()

    for k in range(NBUF - 1):
        launch_gather(k, k)

    def step(i, k):
        bb = NBUF * i + k
        cur = rows[k]
        prv = rows[(k + NBUF - 1) % NBUF]

        @pl.when(bb >= 1)
        def _():
            pltpu.make_async_copy(
                prv, out_slice(bb - 1), ssem[(k + NBUF - 1) % NBUF]).wait()

        @pl.when(bb + NBUF - 1 < B)
        def _():
            launch_gather(bb + NBUF - 1, (k + NBUF - 1) % NBUF)

        wait_gather(k)

        @pl.loop(0, F_PER_W)
        def _(r):
            for j in range(VECS_PER_ROW):
                sl = pl.ds(j * LANES, LANES)
                plsc.addupdate(cur.at[r, sl], pos_v[r, sl])

        pltpu.async_copy(cur, out_slice(bb), ssem[k])

    def body(i, carry):
        for k in range(NBUF):
            step(i, k)
        return carry

    lax.fori_loop(0, B // NBUF, body, 0)
    pltpu.make_async_copy(rows[(B - 1) % NBUF], out_slice(B - 1),
                          ssem[(B - 1) % NBUF]).wait()


@jax.jit
def kernel(x, token_table, pos_table):
    xp = x.reshape(B, NUM_WORKERS, F_PER_W).transpose(1, 0, 2).reshape(-1)
    mesh = plsc.VectorSubcoreMesh(core_axis_name="c", subcore_axis_name="s")
    out = pl.kernel(
        _emb_body,
        out_type=jax.ShapeDtypeStruct((B * D, D), jnp.float32),
        mesh=mesh,
        scratch_types=[
            pltpu.VMEM((F_PER_W, D), jnp.float32),
            pltpu.VMEM((IDX_PER_W,), jnp.int32),
            pltpu.VMEM_SHARED((NUM_PATCHES, D), jnp.float32),
            pltpu.VMEM((F_PER_W, D), jnp.float32),
            pltpu.VMEM((F_PER_W, D), jnp.float32),
            pltpu.SemaphoreType.DMA,
            pltpu.SemaphoreType.DMA,
            pltpu.SemaphoreType.DMA,
            pltpu.SemaphoreType.DMA,
            pltpu.SemaphoreType.DMA,
            pltpu.SemaphoreType.DMA,
        ],
    )(xp, token_table, pos_table)
    return out.reshape(B, D, D)


# speedup vs baseline: 1.1467x; 1.0708x over previous
import jax
import jax.numpy as jnp
from jax import lax
from jax.experimental import pallas as pl
from jax.experimental.pallas import tpu as pltpu
from jax.experimental.pallas import tpu_sc as plsc

NUM_PATCHES = 1024
D = 768
B = 64
NUM_WORKERS = 32
F_PER_W = D // NUM_WORKERS
LANES = 16
VECS_PER_ROW = D // LANES
IDX_PER_W = B * F_PER_W
RB = 8
NBLK = IDX_PER_W // RB
NBUF = 6


def _emb_body(x_hbm, tok_hbm, pos_hbm, out_hbm, pos_v, idx_all, *bufs):
    rows = bufs[:NBUF]
    gsem = bufs[NBUF:2 * NBUF]
    ssem = bufs[2 * NBUF:]

    c = lax.axis_index("c")
    s = lax.axis_index("s")
    wid = s * 2 + c
    f0 = wid * F_PER_W

    pltpu.sync_copy(pos_hbm.at[pl.ds(f0, F_PER_W)], pos_v)
    pltpu.sync_copy(x_hbm.at[pl.ds(wid * IDX_PER_W, IDX_PER_W)], idx_all)

    def idx_slice(m):
        return idx_all.at[pl.ds(m * RB, RB)]

    def out_slice(m):
        n0 = m * RB
        b = n0 // F_PER_W
        r0 = n0 % F_PER_W
        return out_hbm.at[pl.ds(b * D + f0 + r0, RB)], r0

    for k in range(NBUF - 1):
        pltpu.async_copy(tok_hbm.at[idx_slice(k)], rows[k], gsem[k])

    def step(i, k):
        m = NBUF * i + k
        cur = rows[k]
        prv = rows[(k + NBUF - 1) % NBUF]

        @pl.when(m >= 1)
        def _():
            dst, _ = out_slice(m - 1)
            pltpu.make_async_copy(prv, dst, ssem[(k + NBUF - 1) % NBUF]).wait()

        @pl.when(m + NBUF - 1 < NBLK)
        def _():
            pltpu.async_copy(tok_hbm.at[idx_slice(m + NBUF - 1)], prv,
                             gsem[(k + NBUF - 1) % NBUF])

        pltpu.make_async_copy(tok_hbm.at[idx_slice(m)], cur, gsem[k]).wait()

        dst, r0 = out_slice(m)

        @pl.loop(0, RB)
        def _(r):
            for j in range(VECS_PER_ROW):
                sl = pl.ds(j * LANES, LANES)
                plsc.addupdate(cur.at[r, sl], pos_v[r0 + r, sl])

        pltpu.async_copy(cur, dst, ssem[k])

    def body(i, carry):
        for k in range(NBUF):
            step(i, k)
        return carry

    lax.fori_loop(0, NBLK // NBUF, body, 0)
    dst_last, _ = out_slice(NBLK - 1)
    pltpu.make_async_copy(rows[(NBLK - 1) % NBUF], dst_last,
                          ssem[(NBLK - 1) % NBUF]).wait()


@jax.jit
def kernel(x, token_table, pos_table):
    xp = x.reshape(B, NUM_WORKERS, F_PER_W).transpose(1, 0, 2).reshape(-1)
    mesh = plsc.VectorSubcoreMesh(core_axis_name="c", subcore_axis_name="s")
    scratch = (
        [pltpu.VMEM((F_PER_W, D), jnp.float32),
         pltpu.VMEM((IDX_PER_W,), jnp.int32)]
        + [pltpu.VMEM((RB, D), jnp.float32) for _ in range(NBUF)]
        + [pltpu.SemaphoreType.DMA for _ in range(2 * NBUF)]
    )
    out = pl.kernel(
        _emb_body,
        out_type=jax.ShapeDtypeStruct((B * D, D), jnp.float32),
        mesh=mesh,
        scratch_types=scratch,
    )(xp, token_table, pos_table)
    return out.reshape(B, D, D)


# speedup vs baseline: 1.3275x; 1.1577x over previous
import jax
import jax.numpy as jnp
from jax import lax
from jax.experimental import pallas as pl
from jax.experimental.pallas import tpu as pltpu
from jax.experimental.pallas import tpu_sc as plsc

NUM_PATCHES = 1024
D = 768
B = 64
NUM_WORKERS = 32
F_PER_W = D // NUM_WORKERS
LANES = 16
VECS_PER_ROW = D // LANES
IDX_PER_W = B * F_PER_W
RB = 24
NBLK = IDX_PER_W // RB
NBUF = 2
SUB = 8


def _emb_body(x_hbm, tok_hbm, pos_hbm, out_hbm, pos_v, idx_all, *bufs):
    rows = bufs[:NBUF]
    gsem = bufs[NBUF:2 * NBUF]
    ssem = bufs[2 * NBUF:]

    c = lax.axis_index("c")
    s = lax.axis_index("s")
    wid = s * 2 + c
    f0 = wid * F_PER_W

    pltpu.sync_copy(pos_hbm.at[pl.ds(f0, F_PER_W)], pos_v)
    pltpu.sync_copy(x_hbm.at[pl.ds(wid * IDX_PER_W, IDX_PER_W)], idx_all)

    def idx_slice(m):
        return idx_all.at[pl.ds(m * RB, RB)]

    def out_slice(m):
        n0 = m * RB
        b = n0 // F_PER_W
        r0 = n0 % F_PER_W
        return out_hbm.at[pl.ds(b * D + f0 + r0, RB)], r0

    for k in range(NBUF - 1):
        pltpu.async_copy(tok_hbm.at[idx_slice(k)], rows[k], gsem[k])

    def step(i, k):
        m = NBUF * i + k
        cur = rows[k]
        prv = rows[(k + NBUF - 1) % NBUF]

        @pl.when(m >= 1)
        def _():
            dst, _ = out_slice(m - 1)
            pltpu.make_async_copy(prv, dst, ssem[(k + NBUF - 1) % NBUF]).wait()

        @pl.when(m + NBUF - 1 < NBLK)
        def _():
            pltpu.async_copy(tok_hbm.at[idx_slice(m + NBUF - 1)], prv,
                             gsem[(k + NBUF - 1) % NBUF])

        pltpu.make_async_copy(tok_hbm.at[idx_slice(m)], cur, gsem[k]).wait()

        n0 = m * RB
        b = n0 // F_PER_W
        r0 = n0 % F_PER_W
        for h in range(RB // SUB):
            @pl.loop(h * SUB, (h + 1) * SUB)
            def _(r):
                for j in range(VECS_PER_ROW):
                    sl = pl.ds(j * LANES, LANES)
                    plsc.addupdate(cur.at[r, sl], pos_v[r0 + r, sl])

            pltpu.async_copy(
                cur.at[pl.ds(h * SUB, SUB)],
                out_hbm.at[pl.ds(b * D + f0 + r0 + h * SUB, SUB)], ssem[k])

    def body(i, carry):
        for k in range(NBUF):
            step(i, k)
        return carry

    lax.fori_loop(0, NBLK // NBUF, body, 0)
    dst_last, _ = out_slice(NBLK - 1)
    pltpu.make_async_copy(rows[(NBLK - 1) % NBUF], dst_last,
                          ssem[(NBLK - 1) % NBUF]).wait()


@jax.jit
def kernel(x, token_table, pos_table):
    xp = x.reshape(B, NUM_WORKERS, F_PER_W).transpose(1, 0, 2).reshape(-1)
    mesh = plsc.VectorSubcoreMesh(core_axis_name="c", subcore_axis_name="s")
    scratch = (
        [pltpu.VMEM((F_PER_W, D), jnp.float32),
         pltpu.VMEM((IDX_PER_W,), jnp.int32)]
        + [pltpu.VMEM((RB, D), jnp.float32) for _ in range(NBUF)]
        + [pltpu.SemaphoreType.DMA for _ in range(2 * NBUF)]
    )
    out = pl.kernel(
        _emb_body,
        out_type=jax.ShapeDtypeStruct((B * D, D), jnp.float32),
        mesh=mesh,
        scratch_types=scratch,
    )(xp, token_table, pos_table)
    return out.reshape(B, D, D)


# speedup vs baseline: 1.3762x; 1.0366x over previous
import jax
import jax.numpy as jnp
from jax import lax
from jax.experimental import pallas as pl
from jax.experimental.pallas import tpu as pltpu
from jax.experimental.pallas import tpu_sc as plsc

NUM_PATCHES = 1024
D = 768
B = 64
NUM_WORKERS = 32
F_PER_W = D // NUM_WORKERS
LANES = 16
VECS_PER_ROW = D // LANES
IDX_PER_W = B * F_PER_W
RB = 24
NBLK = IDX_PER_W // RB
NBUF = 2
SUB = 8


def _emb_body(x_hbm, tok_hbm, pos_hbm, out_hbm, pos_v, idx_all, *bufs):
    rows = bufs[:NBUF]
    gsem = bufs[NBUF:2 * NBUF]
    ssem = bufs[2 * NBUF:]

    c = lax.axis_index("c")
    s = lax.axis_index("s")
    wid = s * 2 + c
    f0 = wid * F_PER_W

    pltpu.sync_copy(pos_hbm.at[pl.ds(f0, F_PER_W)], pos_v)
    pltpu.sync_copy(x_hbm.at[pl.ds(wid * IDX_PER_W, IDX_PER_W)], idx_all)

    def idx_slice(m):
        return idx_all.at[pl.ds(m * RB, RB)]

    def out_slice(m):
        n0 = m * RB
        b = n0 // F_PER_W
        r0 = n0 % F_PER_W
        return out_hbm.at[pl.ds(b * D + f0 + r0, RB)], r0

    def launch_gather(m, k):
        for h in range(RB // SUB):
            pltpu.async_copy(
                tok_hbm.at[idx_all.at[pl.ds(m * RB + h * SUB, SUB)]],
                rows[k].at[pl.ds(h * SUB, SUB)], gsem[k])

    for k in range(NBUF - 1):
        launch_gather(k, k)

    def step(i, k):
        m = NBUF * i + k
        cur = rows[k]
        prv = rows[(k + NBUF - 1) % NBUF]

        @pl.when(m >= 1)
        def _():
            dst, _ = out_slice(m - 1)
            pltpu.make_async_copy(prv, dst, ssem[(k + NBUF - 1) % NBUF]).wait()

        @pl.when(m + NBUF - 1 < NBLK)
        def _():
            launch_gather(m + NBUF - 1, (k + NBUF - 1) % NBUF)

        n0 = m * RB
        b = n0 // F_PER_W
        r0 = n0 % F_PER_W
        for h in range(RB // SUB):
            pltpu.make_async_copy(
                tok_hbm.at[idx_all.at[pl.ds(h * SUB, SUB)]],
                cur.at[pl.ds(h * SUB, SUB)], gsem[k]).wait()

            @pl.loop(h * SUB, (h + 1) * SUB)
            def _(r):
                for j in range(VECS_PER_ROW):
                    sl = pl.ds(j * LANES, LANES)
                    plsc.addupdate(cur.at[r, sl], pos_v[r0 + r, sl])

            pltpu.async_copy(
                cur.at[pl.ds(h * SUB, SUB)],
                out_hbm.at[pl.ds(b * D + f0 + r0 + h * SUB, SUB)], ssem[k])

    def body(i, carry):
        for k in range(NBUF):
            step(i, k)
        return carry

    lax.fori_loop(0, NBLK // NBUF, body, 0)
    dst_last, _ = out_slice(NBLK - 1)
    pltpu.make_async_copy(rows[(NBLK - 1) % NBUF], dst_last,
                          ssem[(NBLK - 1) % NBUF]).wait()


@jax.jit
def kernel(x, token_table, pos_table):
    xp = x.reshape(B, NUM_WORKERS, F_PER_W).transpose(1, 0, 2).reshape(-1)
    mesh = plsc.VectorSubcoreMesh(core_axis_name="c", subcore_axis_name="s")
    scratch = (
        [pltpu.VMEM((F_PER_W, D), jnp.float32),
         pltpu.VMEM((IDX_PER_W,), jnp.int32)]
        + [pltpu.VMEM((RB, D), jnp.float32) for _ in range(NBUF)]
        + [pltpu.SemaphoreType.DMA for _ in range(2 * NBUF)]
    )
    out = pl.kernel(
        _emb_body,
        out_type=jax.ShapeDtypeStruct((B * D, D), jnp.float32),
        mesh=mesh,
        scratch_types=scratch,
    )(xp, token_table, pos_table)
    return out.reshape(B, D, D)
